# Initial kernel scaffold; baseline (speedup 1.0000x reference)
import functools

import jax
import jax.numpy as jnp
from jax import lax
from jax.experimental import pallas as pl
from jax.experimental.pallas import tpu as pltpu

_N = 10000
_E = 320000
_F = 128
_NB = 8
_RH = 64
_OUT_S = 64
_OUT_V = 32
_MAX_R = 5.0
_NUM_NEIGH = 32.0

_BLK_E = 1280


def _edge_dense_kernel(len_ref, vecn_ref, as_ref, xvs_ref,
                       W1_ref, b1_ref, W2s_ref, W2v_ref,
                       msgs_ref, msgv_ref):
    lengths = len_ref[...]
    centers = lax.broadcasted_iota(jnp.float32, (1, _NB), 1) * (_MAX_R / (_NB - 1))
    width = _MAX_R / _NB
    diff = (lengths - centers) * (1.0 / width)
    basis = jnp.exp(-(diff * diff))
    u = jnp.clip(1.0 - lengths * (1.0 / _MAX_R), 0.0, 1.0)
    env = u * u * (3.0 - 2.0 * u)
    pre = jnp.dot(basis, W1_ref[...], preferred_element_type=jnp.float32) + b1_ref[...]
    h = pre * jax.nn.sigmoid(pre)
    w_s = jnp.dot(h, W2s_ref[...], preferred_element_type=jnp.float32) * env
    w_v = jnp.dot(h, W2v_ref[...], preferred_element_type=jnp.float32) * env
    msgs_ref[...] = as_ref[...] * w_s
    xvw = xvs_ref[...] * w_v
    vn = vecn_ref[...]
    for k in range(3):
        msgv_ref[:, k * _OUT_V:(k + 1) * _OUT_V] = xvw * vn[:, k:k + 1]


def kernel(node_attrs, positions, edge_index, shifts, W1, b1, W2s, W2v, Wv, Ws):
    sender = edge_index[0]
    receiver = edge_index[1]
    vectors = positions[receiver] - positions[sender] + shifts
    lengths = jnp.sqrt(jnp.sum(vectors * vectors, axis=-1, keepdims=True))
    vec_n = vectors / (lengths + 1e-9)

    xv = node_attrs @ Wv
    a_s = node_attrs[sender]
    xv_s = xv[sender]

    grid = (_E // _BLK_E,)
    eb = lambda i: (i, 0)
    msg_s, msg_v = pl.pallas_call(
        _edge_dense_kernel,
        grid=grid,
        in_specs=[
            pl.BlockSpec((_BLK_E, 1), eb),
            pl.BlockSpec((_BLK_E, 3), eb),
            pl.BlockSpec((_BLK_E, _F), eb),
            pl.BlockSpec((_BLK_E, _OUT_V), eb),
            pl.BlockSpec((_NB, _RH), lambda i: (0, 0)),
            pl.BlockSpec((1, _RH), lambda i: (0, 0)),
            pl.BlockSpec((_RH, _F), lambda i: (0, 0)),
            pl.BlockSpec((_RH, _OUT_V), lambda i: (0, 0)),
        ],
        out_specs=[
            pl.BlockSpec((_BLK_E, _F), eb),
            pl.BlockSpec((_BLK_E, 3 * _OUT_V), eb),
        ],
        out_shape=[
            jax.ShapeDtypeStruct((_E, _F), jnp.float32),
            jax.ShapeDtypeStruct((_E, 3 * _OUT_V), jnp.float32),
        ],
    )(lengths, vec_n, a_s, xv_s, W1, b1.reshape(1, _RH), W2s, W2v)

    inv_sqrt = 1.0 / jnp.sqrt(_NUM_NEIGH)
    agg_s = jax.ops.segment_sum(msg_s, receiver, num_segments=_N) * inv_sqrt
    out_s = agg_s @ Ws
    agg_v_km = jax.ops.segment_sum(msg_v, receiver, num_segments=_N) * inv_sqrt
    agg_v = agg_v_km.reshape(_N, 3, _OUT_V).transpose(0, 2, 1).reshape(_N, 3 * _OUT_V)
    return jnp.concatenate([out_s, agg_v], axis=1)


# speedup vs baseline: 6.7188x; 6.7188x over previous
import functools

import jax
import jax.numpy as jnp
from jax import lax
from jax.experimental import pallas as pl
from jax.experimental.pallas import tpu as pltpu

_N = 10000
_E = 320000
_F = 128
_NB = 8
_RH = 64
_OUT_S = 64
_OUT_V = 32
_MAX_R = 5.0
_NUM_NEIGH = 32.0

_BLK_E = 1280


def _edge_dense_kernel(len_ref, vecn_ref, as_ref, xvs_ref,
                       W1_ref, b1_ref, W2s_ref, W2v_ref,
                       msgs_ref, msgv_ref):
    lengths = len_ref[...]
    centers = lax.broadcasted_iota(jnp.int32, (1, _NB), 1).astype(jnp.float32) * (_MAX_R / (_NB - 1))
    width = _MAX_R / _NB
    diff = (lengths - centers) * (1.0 / width)
    basis = jnp.exp(-(diff * diff))
    u = jnp.clip(1.0 - lengths * (1.0 / _MAX_R), 0.0, 1.0)
    env = u * u * (3.0 - 2.0 * u)
    pre = jnp.dot(basis, W1_ref[...], preferred_element_type=jnp.float32) + b1_ref[...]
    h = pre * jax.nn.sigmoid(pre)
    w_s = jnp.dot(h, W2s_ref[...], preferred_element_type=jnp.float32) * env
    w_v = jnp.dot(h, W2v_ref[...], preferred_element_type=jnp.float32) * env
    msgs_ref[...] = as_ref[...] * w_s
    xvw = xvs_ref[...] * w_v
    vn = vecn_ref[...]
    for k in range(3):
        msgv_ref[:, k * _OUT_V:(k + 1) * _OUT_V] = xvw * vn[:, k:k + 1]


def kernel(node_attrs, positions, edge_index, shifts, W1, b1, W2s, W2v, Wv, Ws):
    sender = edge_index[0]
    receiver = edge_index[1]
    vectors = positions[receiver] - positions[sender] + shifts
    lengths = jnp.sqrt(jnp.sum(vectors * vectors, axis=-1, keepdims=True))
    vec_n = vectors / (lengths + 1e-9)

    xv = node_attrs @ Wv
    a_s = node_attrs[sender]
    xv_s = xv[sender]

    grid = (_E // _BLK_E,)
    eb = lambda i: (i, 0)
    msg_s, msg_v = pl.pallas_call(
        _edge_dense_kernel,
        grid=grid,
        in_specs=[
            pl.BlockSpec((_BLK_E, 1), eb),
            pl.BlockSpec((_BLK_E, 3), eb),
            pl.BlockSpec((_BLK_E, _F), eb),
            pl.BlockSpec((_BLK_E, _OUT_V), eb),
            pl.BlockSpec((_NB, _RH), lambda i: (0, 0)),
            pl.BlockSpec((1, _RH), lambda i: (0, 0)),
            pl.BlockSpec((_RH, _F), lambda i: (0, 0)),
            pl.BlockSpec((_RH, _OUT_V), lambda i: (0, 0)),
        ],
        out_specs=[
            pl.BlockSpec((_BLK_E, _F), eb),
            pl.BlockSpec((_BLK_E, 3 * _OUT_V), eb),
        ],
        out_shape=[
            jax.ShapeDtypeStruct((_E, _F), jnp.float32),
            jax.ShapeDtypeStruct((_E, 3 * _OUT_V), jnp.float32),
        ],
    )(lengths, vec_n, a_s, xv_s, W1, b1.reshape(1, _RH), W2s, W2v)

    inv_sqrt = 1.0 / jnp.sqrt(_NUM_NEIGH)
    agg_s = jax.ops.segment_sum(msg_s, receiver, num_segments=_N) * inv_sqrt
    out_s = agg_s @ Ws
    agg_v_km = jax.ops.segment_sum(msg_v, receiver, num_segments=_N) * inv_sqrt
    agg_v = agg_v_km.reshape(_N, 3, _OUT_V).transpose(0, 2, 1).reshape(_N, 3 * _OUT_V)
    return jnp.concatenate([out_s, agg_v], axis=1)


# speedup vs baseline: 26.5886x; 3.9573x over previous
import functools

import jax
import jax.numpy as jnp
from jax import lax
from jax.experimental import pallas as pl
from jax.experimental.pallas import tpu as pltpu
from jax.experimental.pallas import tpu_sc as plsc

_N = 10000
_NP = 10240
_E = 320000
_F = 128
_NB = 8
_RH = 64
_OUT_S = 64
_OUT_V = 32
_MAX_R = 5.0
_NUM_NEIGH = 32.0
_INV_SQRT = 1.0 / float(_NUM_NEIGH) ** 0.5

_NC, _NS = 2, 16
_NW = _NC * _NS
_EW = _E // _NW
_CH = 128
_NCHUNKS = _E // _CH
_CHW = _NCHUNKS // _NW
_CH_REM = _NCHUNKS - _CHW * _NW
_ROWS_T = _NP // _NS

_BLK_E = 1280
_BLK_N = 2000
_GCH = 1000

_SC_PARAMS = pltpu.CompilerParams(use_tc_tiling_on_sc=False)


def _sc_mesh():
    return plsc.VectorSubcoreMesh(core_axis_name="c", subcore_axis_name="s",
                                  num_cores=_NC, num_subcores=_NS)


def _prep_kernel(attr_ref, pos_ref, Wv_ref, xv_ref, p8_ref):
    xv_ref[...] = jnp.dot(attr_ref[...], Wv_ref[...],
                          preferred_element_type=jnp.float32)
    p8_ref[:, 0:3] = pos_ref[...]
    p8_ref[:, 3:8] = jnp.zeros((_BLK_N, 5), jnp.float32)


def _run_prep(node_attrs, positions, Wv):
    nb = lambda i: (i, 0)
    return pl.pallas_call(
        _prep_kernel,
        grid=(_N // _BLK_N,),
        in_specs=[
            pl.BlockSpec((_BLK_N, _F), nb),
            pl.BlockSpec((_BLK_N, 3), nb),
            pl.BlockSpec((_F, _OUT_V), lambda i: (0, 0)),
        ],
        out_specs=[
            pl.BlockSpec((_BLK_N, _OUT_V), nb),
            pl.BlockSpec((_BLK_N, 8), nb),
        ],
        out_shape=[
            jax.ShapeDtypeStruct((_N, _OUT_V), jnp.float32),
            jax.ShapeDtypeStruct((_N, 8), jnp.float32),
        ],
    )(node_attrs, positions, Wv)


def _geo_body(p8_hbm, send_hbm, recv_hbm, ps_hbm, pr_hbm,
              idx_v, ps_v, pr_v, sem1, sem2):
    c = lax.axis_index("c")
    s = lax.axis_index("s")
    wid = s * _NC + c

    def chunk(i, carry):
        base = wid * _EW + i * _GCH
        pltpu.sync_copy(send_hbm.at[pl.ds(base, _GCH)], idx_v)
        pltpu.async_copy(p8_hbm.at[idx_v], ps_v, sem1).wait()
        pltpu.sync_copy(ps_v, ps_hbm.at[pl.ds(base, _GCH)])
        pltpu.sync_copy(recv_hbm.at[pl.ds(base, _GCH)], idx_v)
        pltpu.async_copy(p8_hbm.at[idx_v], pr_v, sem2).wait()
        pltpu.sync_copy(pr_v, pr_hbm.at[pl.ds(base, _GCH)])
        return carry

    lax.fori_loop(0, _EW // _GCH, chunk, 0)


def _run_geo(p8, sender, receiver):
    f = pl.kernel(
        _geo_body,
        out_type=[
            jax.ShapeDtypeStruct((_E, 8), jnp.float32),
            jax.ShapeDtypeStruct((_E, 8), jnp.float32),
        ],
        mesh=_sc_mesh(),
        compiler_params=_SC_PARAMS,
        scratch_types=[
            pltpu.VMEM((_GCH,), jnp.int32),
            pltpu.VMEM((_GCH, 8), jnp.float32),
            pltpu.VMEM((_GCH, 8), jnp.float32),
            pltpu.SemaphoreType.DMA,
            pltpu.SemaphoreType.DMA,
        ],
    )
    return f(p8, sender, receiver)


def _edge_kernel(ps_ref, pr_ref, W1_ref, b1_ref, W2s_ref, W2v_ref,
                 ws_ref, wvn_ref):
    d = pr_ref[:, 0:3] - ps_ref[:, 0:3]
    lengths = jnp.sqrt(jnp.sum(d * d, axis=1, keepdims=True))
    vec_n = d / (lengths + 1e-9)
    centers = lax.broadcasted_iota(jnp.int32, (1, _NB), 1).astype(jnp.float32) \
        * (_MAX_R / (_NB - 1))
    width = _MAX_R / _NB
    diff = (lengths - centers) * (1.0 / width)
    basis = jnp.exp(-(diff * diff))
    u = jnp.clip(1.0 - lengths * (1.0 / _MAX_R), 0.0, 1.0)
    env = u * u * (3.0 - 2.0 * u)
    pre = jnp.dot(basis, W1_ref[...], preferred_element_type=jnp.float32) + b1_ref[...]
    h = pre * jax.nn.sigmoid(pre)
    ws_ref[...] = jnp.dot(h, W2s_ref[...], preferred_element_type=jnp.float32) * env
    wvn_ref[:, 0:_OUT_V] = jnp.dot(h, W2v_ref[...],
                                   preferred_element_type=jnp.float32) * env
    wvn_ref[:, _OUT_V:_OUT_V + 3] = vec_n
    wvn_ref[:, _OUT_V + 3:48] = jnp.zeros((_BLK_E, 13), jnp.float32)


def _run_edge_dense(ps, pr, W1, b1, W2s, W2v):
    eb = lambda i: (i, 0)
    wb = lambda i: (0, 0)
    return pl.pallas_call(
        _edge_kernel,
        grid=(_E // _BLK_E,),
        in_specs=[
            pl.BlockSpec((_BLK_E, 8), eb),
            pl.BlockSpec((_BLK_E, 8), eb),
            pl.BlockSpec((_NB, _RH), wb),
            pl.BlockSpec((1, _RH), wb),
            pl.BlockSpec((_RH, _F), wb),
            pl.BlockSpec((_RH, _OUT_V), wb),
        ],
        out_specs=[
            pl.BlockSpec((_BLK_E, _F), eb),
            pl.BlockSpec((_BLK_E, 48), eb),
        ],
        out_shape=[
            jax.ShapeDtypeStruct((_E, _F), jnp.float32),
            jax.ShapeDtypeStruct((_E, 48), jnp.float32),
        ],
    )(ps, pr, W1, b1.reshape(1, _RH), W2s, W2v)


def _scat_s_body(attr_hbm, ws_hbm, send_hbm, recv_hbm, zero_hbm, out_hbm,
                 idxs_v, idxr_v, a_v, w_v, accum, sem):
    c = lax.axis_index("c")
    s = lax.axis_index("s")
    wid = s * _NC + c
    row0 = s * _ROWS_T
    pltpu.sync_copy(zero_hbm, accum.at[pl.ds(row0, _ROWS_T)])
    plsc.subcore_barrier()
    nch = jnp.where(wid < _CH_REM, _CHW + 1, _CHW)

    def chunk(i, carry):
        base = (wid + i * _NW) * _CH
        pltpu.sync_copy(send_hbm.at[pl.ds(base, _CH)], idxs_v)
        gat = pltpu.async_copy(attr_hbm.at[idxs_v], a_v, sem)
        pltpu.sync_copy(ws_hbm.at[pl.ds(base, _CH)], w_v)
        pltpu.sync_copy(recv_hbm.at[pl.ds(base, _CH)], idxr_v)
        gat.wait()

        def rowbody(e, _):
            for k in range(_F // 16):
                sl = pl.ds(k * 16, 16)
                a_v[e, sl] = a_v[e, sl] * w_v[e, sl]
            return 0

        lax.fori_loop(0, _CH, rowbody, 0)
        pltpu.sync_copy(a_v, accum.at[idxr_v], add=True)
        return carry

    lax.fori_loop(0, nch, chunk, 0)
    plsc.subcore_barrier()
    pltpu.sync_copy(accum.at[pl.ds(row0, _ROWS_T)],
                    out_hbm.at[c, pl.ds(row0, _ROWS_T)])


def _run_scat_s(node_attrs, ws, sender, receiver, zero_s):
    f = pl.kernel(
        _scat_s_body,
        out_type=jax.ShapeDtypeStruct((_NC, _NP, _F), jnp.float32),
        mesh=_sc_mesh(),
        compiler_params=_SC_PARAMS,
        scratch_types=[
            pltpu.VMEM((_CH,), jnp.int32),
            pltpu.VMEM((_CH,), jnp.int32),
            pltpu.VMEM((_CH, _F), jnp.float32),
            pltpu.VMEM((_CH, _F), jnp.float32),
            pltpu.VMEM_SHARED((_NP, _F), jnp.float32),
            pltpu.SemaphoreType.DMA,
        ],
    )
    return f(node_attrs, ws, sender, receiver, zero_s)


def _scat_v_body(xv_hbm, wvn_hbm, send_hbm, recv_hbm, zero_hbm, out_hbm,
                 idxs_v, idxr_v, x_v, wv_v, m_v, accum, sem):
    c = lax.axis_index("c")
    s = lax.axis_index("s")
    wid = s * _NC + c
    row0 = s * _ROWS_T
    pltpu.sync_copy(zero_hbm, accum.at[pl.ds(row0, _ROWS_T)])

    def zrow(e, _):
        m_v[e, pl.ds(96, 16)] = jnp.zeros((16,), jnp.float32)
        m_v[e, pl.ds(112, 16)] = jnp.zeros((16,), jnp.float32)
        return 0

    lax.fori_loop(0, _CH, zrow, 0)
    plsc.subcore_barrier()
    nch = jnp.where(wid < _CH_REM, _CHW + 1, _CHW)

    def chunk(i, carry):
        base = (wid + i * _NW) * _CH
        pltpu.sync_copy(send_hbm.at[pl.ds(base, _CH)], idxs_v)
        gat = pltpu.async_copy(xv_hbm.at[idxs_v], x_v, sem)
        pltpu.sync_copy(wvn_hbm.at[pl.ds(base, _CH)], wv_v)
        pltpu.sync_copy(recv_hbm.at[pl.ds(base, _CH)], idxr_v)
        gat.wait()

        def rowbody(e, _):
            s0 = pl.ds(0, 16)
            s1 = pl.ds(16, 16)
            x0 = x_v[e, s0] * wv_v[e, s0]
            x1 = x_v[e, s1] * wv_v[e, s1]
            vn = wv_v[e, pl.ds(_OUT_V, 16)]
            vn0 = vn[0]
            vn1 = vn[1]
            vn2 = vn[2]
            m_v[e, pl.ds(0, 16)] = x0 * vn0
            m_v[e, pl.ds(16, 16)] = x1 * vn0
            m_v[e, pl.ds(32, 16)] = x0 * vn1
            m_v[e, pl.ds(48, 16)] = x1 * vn1
            m_v[e, pl.ds(64, 16)] = x0 * vn2
            m_v[e, pl.ds(80, 16)] = x1 * vn2
            return 0

        lax.fori_loop(0, _CH, rowbody, 0)
        pltpu.sync_copy(m_v, accum.at[idxr_v], add=True)
        return carry

    lax.fori_loop(0, nch, chunk, 0)
    plsc.subcore_barrier()
    pltpu.sync_copy(accum.at[pl.ds(row0, _ROWS_T)],
                    out_hbm.at[c, pl.ds(row0, _ROWS_T)])


def _run_scat_v(xv, wvn, sender, receiver, zero_s):
    f = pl.kernel(
        _scat_v_body,
        out_type=jax.ShapeDtypeStruct((_NC, _NP, _F), jnp.float32),
        mesh=_sc_mesh(),
        compiler_params=_SC_PARAMS,
        scratch_types=[
            pltpu.VMEM((_CH,), jnp.int32),
            pltpu.VMEM((_CH,), jnp.int32),
            pltpu.VMEM((_CH, _OUT_V), jnp.float32),
            pltpu.VMEM((_CH, 48), jnp.float32),
            pltpu.VMEM((_CH, _F), jnp.float32),
            pltpu.VMEM_SHARED((_NP, _F), jnp.float32),
            pltpu.SemaphoreType.DMA,
        ],
    )
    return f(xv, wvn, sender, receiver, zero_s)


def _final_kernel(s0_ref, s1_ref, v0_ref, v1_ref, Ws_ref, outs_ref, aggv_ref):
    agg_s = (s0_ref[0] + s1_ref[0]) * _INV_SQRT
    outs_ref[...] = jnp.dot(agg_s, Ws_ref[...],
                            preferred_element_type=jnp.float32)
    v = (v0_ref[0] + v1_ref[0]) * _INV_SQRT
    aggv_ref[...] = v[:, 0:3 * _OUT_V]


def _run_final(s_part, v_part, Ws):
    return pl.pallas_call(
        _final_kernel,
        grid=(_N // _BLK_N,),
        in_specs=[
            pl.BlockSpec((1, _BLK_N, _F), lambda i: (0, i, 0)),
            pl.BlockSpec((1, _BLK_N, _F), lambda i: (1, i, 0)),
            pl.BlockSpec((1, _BLK_N, _F), lambda i: (0, i, 0)),
            pl.BlockSpec((1, _BLK_N, _F), lambda i: (1, i, 0)),
            pl.BlockSpec((_F, _OUT_S), lambda i: (0, 0)),
        ],
        out_specs=[
            pl.BlockSpec((_BLK_N, _OUT_S), lambda i: (i, 0)),
            pl.BlockSpec((_BLK_N, 3 * _OUT_V), lambda i: (i, 0)),
        ],
        out_shape=[
            jax.ShapeDtypeStruct((_N, _OUT_S), jnp.float32),
            jax.ShapeDtypeStruct((_N, 3 * _OUT_V), jnp.float32),
        ],
    )(s_part, s_part, v_part, v_part, Ws)


def kernel(node_attrs, positions, edge_index, shifts, W1, b1, W2s, W2v, Wv, Ws):
    sender = edge_index[0]
    receiver = edge_index[1]
    zero_s = jnp.zeros((_ROWS_T, _F), jnp.float32)

    xv, p8 = _run_prep(node_attrs, positions, Wv)
    ps, pr = _run_geo(p8, sender, receiver)
    ws, wvn = _run_edge_dense(ps, pr, W1, b1, W2s, W2v)
    s_part = _run_scat_s(node_attrs, ws, sender, receiver, zero_s)
    v_part = _run_scat_v(xv, wvn, sender, receiver, zero_s)
    out_s, aggv_km = _run_final(s_part, v_part, Ws)

    agg_v = aggv_km.reshape(_N, 3, _OUT_V).transpose(0, 2, 1).reshape(_N, 3 * _OUT_V)
    return jnp.concatenate([out_s, agg_v], axis=1)


# speedup vs baseline: 27.8693x; 1.0482x over previous
import functools

import jax
import jax.numpy as jnp
from jax import lax
from jax.experimental import pallas as pl
from jax.experimental.pallas import tpu as pltpu
from jax.experimental.pallas import tpu_sc as plsc

_N = 10000
_NP = 10240
_E = 320000
_F = 128
_NB = 8
_RH = 64
_OUT_S = 64
_OUT_V = 32
_MAX_R = 5.0
_NUM_NEIGH = 32.0
_INV_SQRT = 1.0 / float(_NUM_NEIGH) ** 0.5

_NC, _NS = 2, 16
_NW = _NC * _NS
_EW = _E // _NW
_CH = 128
_NCHUNKS = _E // _CH
_CHW = _NCHUNKS // _NW
_CH_REM = _NCHUNKS - _CHW * _NW
_ROWS_T = _NP // _NS

_BLK_E = 1280
_BLK_N = 2000
_GCH = 1000

_SC_PARAMS = pltpu.CompilerParams(use_tc_tiling_on_sc=False)


def _sc_mesh():
    return plsc.VectorSubcoreMesh(core_axis_name="c", subcore_axis_name="s",
                                  num_cores=_NC, num_subcores=_NS)


def _prep_kernel(attr_ref, pos_ref, Wv_ref, xv_ref, p8_ref):
    xv_ref[...] = jnp.dot(attr_ref[...], Wv_ref[...],
                          preferred_element_type=jnp.float32)
    p8_ref[:, 0:3] = pos_ref[...]
    p8_ref[:, 3:8] = jnp.zeros((_BLK_N, 5), jnp.float32)


def _run_prep(node_attrs, positions, Wv):
    nb = lambda i: (i, 0)
    return pl.pallas_call(
        _prep_kernel,
        grid=(_N // _BLK_N,),
        in_specs=[
            pl.BlockSpec((_BLK_N, _F), nb),
            pl.BlockSpec((_BLK_N, 3), nb),
            pl.BlockSpec((_F, _OUT_V), lambda i: (0, 0)),
        ],
        out_specs=[
            pl.BlockSpec((_BLK_N, _OUT_V), nb),
            pl.BlockSpec((_BLK_N, 8), nb),
        ],
        out_shape=[
            jax.ShapeDtypeStruct((_N, _OUT_V), jnp.float32),
            jax.ShapeDtypeStruct((_N, 8), jnp.float32),
        ],
    )(node_attrs, positions, Wv)


def _geo_body(p8_hbm, send_hbm, recv_hbm, ps_hbm, pr_hbm,
              idx_v, ps_v, pr_v, sem1, sem2):
    c = lax.axis_index("c")
    s = lax.axis_index("s")
    wid = s * _NC + c

    def chunk(i, carry):
        base = wid * _EW + i * _GCH
        pltpu.sync_copy(send_hbm.at[pl.ds(base, _GCH)], idx_v)
        pltpu.async_copy(p8_hbm.at[idx_v], ps_v, sem1).wait()
        pltpu.sync_copy(ps_v, ps_hbm.at[pl.ds(base, _GCH)])
        pltpu.sync_copy(recv_hbm.at[pl.ds(base, _GCH)], idx_v)
        pltpu.async_copy(p8_hbm.at[idx_v], pr_v, sem2).wait()
        pltpu.sync_copy(pr_v, pr_hbm.at[pl.ds(base, _GCH)])
        return carry

    lax.fori_loop(0, _EW // _GCH, chunk, 0)


def _run_geo(p8, sender, receiver):
    f = pl.kernel(
        _geo_body,
        out_type=[
            jax.ShapeDtypeStruct((_E, 8), jnp.float32),
            jax.ShapeDtypeStruct((_E, 8), jnp.float32),
        ],
        mesh=_sc_mesh(),
        compiler_params=_SC_PARAMS,
        scratch_types=[
            pltpu.VMEM((_GCH,), jnp.int32),
            pltpu.VMEM((_GCH, 8), jnp.float32),
            pltpu.VMEM((_GCH, 8), jnp.float32),
            pltpu.SemaphoreType.DMA,
            pltpu.SemaphoreType.DMA,
        ],
    )
    return f(p8, sender, receiver)


def _edge_kernel(ps_ref, pr_ref, W1_ref, b1_ref, W2s_ref, W2v_ref,
                 ws_ref, wvn_ref):
    d = pr_ref[:, 0:3] - ps_ref[:, 0:3]
    lengths = jnp.sqrt(jnp.sum(d * d, axis=1, keepdims=True))
    vec_n = d / (lengths + 1e-9)
    centers = lax.broadcasted_iota(jnp.int32, (1, _NB), 1).astype(jnp.float32) \
        * (_MAX_R / (_NB - 1))
    width = _MAX_R / _NB
    diff = (lengths - centers) * (1.0 / width)
    basis = jnp.exp(-(diff * diff))
    u = jnp.clip(1.0 - lengths * (1.0 / _MAX_R), 0.0, 1.0)
    env = u * u * (3.0 - 2.0 * u)
    pre = jnp.dot(basis, W1_ref[...], preferred_element_type=jnp.float32) + b1_ref[...]
    h = pre * jax.nn.sigmoid(pre)
    ws_ref[...] = jnp.dot(h, W2s_ref[...], preferred_element_type=jnp.float32) * env
    wvn_ref[:, 0:_OUT_V] = jnp.dot(h, W2v_ref[...],
                                   preferred_element_type=jnp.float32) * env
    wvn_ref[:, _OUT_V:_OUT_V + 3] = vec_n
    wvn_ref[:, _OUT_V + 3:48] = jnp.zeros((_BLK_E, 13), jnp.float32)


def _run_edge_dense(ps, pr, W1, b1, W2s, W2v):
    eb = lambda i: (i, 0)
    wb = lambda i: (0, 0)
    return pl.pallas_call(
        _edge_kernel,
        grid=(_E // _BLK_E,),
        in_specs=[
            pl.BlockSpec((_BLK_E, 8), eb),
            pl.BlockSpec((_BLK_E, 8), eb),
            pl.BlockSpec((_NB, _RH), wb),
            pl.BlockSpec((1, _RH), wb),
            pl.BlockSpec((_RH, _F), wb),
            pl.BlockSpec((_RH, _OUT_V), wb),
        ],
        out_specs=[
            pl.BlockSpec((_BLK_E, _F), eb),
            pl.BlockSpec((_BLK_E, 48), eb),
        ],
        out_shape=[
            jax.ShapeDtypeStruct((_E, _F), jnp.float32),
            jax.ShapeDtypeStruct((_E, 48), jnp.float32),
        ],
    )(ps, pr, W1, b1.reshape(1, _RH), W2s, W2v)


def _scat_s_body(attr_hbm, ws_hbm, send_hbm, recv_hbm, zero_hbm, out_hbm,
                 idxs_v, idxr_v, a_v, w_v, accum, sem, sem2, sem3):
    c = lax.axis_index("c")
    s = lax.axis_index("s")
    wid = s * _NC + c
    row0 = s * _ROWS_T
    pltpu.sync_copy(zero_hbm, accum.at[pl.ds(row0, _ROWS_T)])
    plsc.subcore_barrier()
    nch = jnp.where(wid < _CH_REM, _CHW + 1, _CHW)

    def chunk(i, carry):
        base = (wid + i * _NW) * _CH
        pltpu.sync_copy(send_hbm.at[pl.ds(base, _CH)], idxs_v)
        gat = pltpu.async_copy(attr_hbm.at[idxs_v], a_v, sem)
        lw = pltpu.async_copy(ws_hbm.at[pl.ds(base, _CH)], w_v, sem2)
        li = pltpu.async_copy(recv_hbm.at[pl.ds(base, _CH)], idxr_v, sem3)
        gat.wait()
        lw.wait()
        li.wait()

        def grpbody(g, _):
            for j in range(8):
                e = g * 8 + j
                for k in range(_F // 16):
                    sl = pl.ds(k * 16, 16)
                    a_v[e, sl] = a_v[e, sl] * w_v[e, sl]
            return 0

        lax.fori_loop(0, _CH // 8, grpbody, 0)
        pltpu.sync_copy(a_v, accum.at[idxr_v], add=True)
        return carry

    lax.fori_loop(0, nch, chunk, 0)
    plsc.subcore_barrier()
    pltpu.sync_copy(accum.at[pl.ds(row0, _ROWS_T)],
                    out_hbm.at[c, pl.ds(row0, _ROWS_T)])


def _run_scat_s(node_attrs, ws, sender, receiver, zero_s):
    f = pl.kernel(
        _scat_s_body,
        out_type=jax.ShapeDtypeStruct((_NC, _NP, _F), jnp.float32),
        mesh=_sc_mesh(),
        compiler_params=pltpu.CompilerParams(use_tc_tiling_on_sc=True),
        scratch_types=[
            pltpu.VMEM((_CH,), jnp.int32),
            pltpu.VMEM((_CH,), jnp.int32),
            pltpu.VMEM((_CH, _F), jnp.float32),
            pltpu.VMEM((_CH, _F), jnp.float32),
            pltpu.VMEM_SHARED((_NP, _F), jnp.float32),
            pltpu.SemaphoreType.DMA,
            pltpu.SemaphoreType.DMA,
            pltpu.SemaphoreType.DMA,
        ],
    )
    return f(node_attrs, ws, sender, receiver, zero_s)


def _scat_v_body(xv_hbm, wvn_hbm, send_hbm, recv_hbm, zero_hbm, out_hbm,
                 idxs_v, idxr_v, x_v, wv_v, m_v, accum, sem, sem2, sem3):
    c = lax.axis_index("c")
    s = lax.axis_index("s")
    wid = s * _NC + c
    row0 = s * _ROWS_T
    pltpu.sync_copy(zero_hbm, accum.at[pl.ds(row0, _ROWS_T)])

    def zrow(e, _):
        m_v[e, pl.ds(96, 16)] = jnp.zeros((16,), jnp.float32)
        m_v[e, pl.ds(112, 16)] = jnp.zeros((16,), jnp.float32)
        return 0

    lax.fori_loop(0, _CH, zrow, 0)
    plsc.subcore_barrier()
    nch = jnp.where(wid < _CH_REM, _CHW + 1, _CHW)

    def chunk(i, carry):
        base = (wid + i * _NW) * _CH
        pltpu.sync_copy(send_hbm.at[pl.ds(base, _CH)], idxs_v)
        gat = pltpu.async_copy(xv_hbm.at[idxs_v], x_v, sem)
        lw = pltpu.async_copy(wvn_hbm.at[pl.ds(base, _CH)], wv_v, sem2)
        li = pltpu.async_copy(recv_hbm.at[pl.ds(base, _CH)], idxr_v, sem3)
        gat.wait()
        lw.wait()
        li.wait()

        def grpbody(g, _):
            for j in range(4):
                e = g * 4 + j
                s0 = pl.ds(0, 16)
                s1 = pl.ds(16, 16)
                x0 = x_v[e, s0] * wv_v[e, s0]
                x1 = x_v[e, s1] * wv_v[e, s1]
                vn = wv_v[e, pl.ds(_OUT_V, 16)]
                vn0 = vn[0]
                vn1 = vn[1]
                vn2 = vn[2]
                m_v[e, pl.ds(0, 16)] = x0 * vn0
                m_v[e, pl.ds(16, 16)] = x1 * vn0
                m_v[e, pl.ds(32, 16)] = x0 * vn1
                m_v[e, pl.ds(48, 16)] = x1 * vn1
                m_v[e, pl.ds(64, 16)] = x0 * vn2
                m_v[e, pl.ds(80, 16)] = x1 * vn2
            return 0

        lax.fori_loop(0, _CH // 4, grpbody, 0)
        pltpu.sync_copy(m_v, accum.at[idxr_v], add=True)
        return carry

    lax.fori_loop(0, nch, chunk, 0)
    plsc.subcore_barrier()
    pltpu.sync_copy(accum.at[pl.ds(row0, _ROWS_T)],
                    out_hbm.at[c, pl.ds(row0, _ROWS_T)])


def _run_scat_v(xv, wvn, sender, receiver, zero_s):
    f = pl.kernel(
        _scat_v_body,
        out_type=jax.ShapeDtypeStruct((_NC, _NP, _F), jnp.float32),
        mesh=_sc_mesh(),
        compiler_params=_SC_PARAMS,
        scratch_types=[
            pltpu.VMEM((_CH,), jnp.int32),
            pltpu.VMEM((_CH,), jnp.int32),
            pltpu.VMEM((_CH, _OUT_V), jnp.float32),
            pltpu.VMEM((_CH, 48), jnp.float32),
            pltpu.VMEM((_CH, _F), jnp.float32),
            pltpu.VMEM_SHARED((_NP, _F), jnp.float32),
            pltpu.SemaphoreType.DMA,
            pltpu.SemaphoreType.DMA,
            pltpu.SemaphoreType.DMA,
        ],
    )
    return f(xv, wvn, sender, receiver, zero_s)


def _final_kernel(s0_ref, s1_ref, v0_ref, v1_ref, Ws_ref, outs_ref, aggv_ref):
    agg_s = (s0_ref[0] + s1_ref[0]) * _INV_SQRT
    outs_ref[...] = jnp.dot(agg_s, Ws_ref[...],
                            preferred_element_type=jnp.float32)
    v = (v0_ref[0] + v1_ref[0]) * _INV_SQRT
    aggv_ref[...] = v[:, 0:3 * _OUT_V]


def _run_final(s_part, v_part, Ws):
    return pl.pallas_call(
        _final_kernel,
        grid=(_N // _BLK_N,),
        in_specs=[
            pl.BlockSpec((1, _BLK_N, _F), lambda i: (0, i, 0)),
            pl.BlockSpec((1, _BLK_N, _F), lambda i: (1, i, 0)),
            pl.BlockSpec((1, _BLK_N, _F), lambda i: (0, i, 0)),
            pl.BlockSpec((1, _BLK_N, _F), lambda i: (1, i, 0)),
            pl.BlockSpec((_F, _OUT_S), lambda i: (0, 0)),
        ],
        out_specs=[
            pl.BlockSpec((_BLK_N, _OUT_S), lambda i: (i, 0)),
            pl.BlockSpec((_BLK_N, 3 * _OUT_V), lambda i: (i, 0)),
        ],
        out_shape=[
            jax.ShapeDtypeStruct((_N, _OUT_S), jnp.float32),
            jax.ShapeDtypeStruct((_N, 3 * _OUT_V), jnp.float32),
        ],
    )(s_part, s_part, v_part, v_part, Ws)


def kernel(node_attrs, positions, edge_index, shifts, W1, b1, W2s, W2v, Wv, Ws):
    sender = edge_index[0]
    receiver = edge_index[1]
    zero_s = jnp.zeros((_ROWS_T, _F), jnp.float32)

    xv, p8 = _run_prep(node_attrs, positions, Wv)
    ps, pr = _run_geo(p8, sender, receiver)
    ws, wvn = _run_edge_dense(ps, pr, W1, b1, W2s, W2v)
    s_part = _run_scat_s(node_attrs, ws, sender, receiver, zero_s)
    v_part = _run_scat_v(xv, wvn, sender, receiver, zero_s)
    out_s, aggv_km = _run_final(s_part, v_part, Ws)

    agg_v = aggv_km.reshape(_N, 3, _OUT_V).transpose(0, 2, 1).reshape(_N, 3 * _OUT_V)
    return jnp.concatenate([out_s, agg_v], axis=1)


# speedup vs baseline: 31.3975x; 1.1266x over previous
import functools

import jax
import jax.numpy as jnp
from jax import lax
from jax.experimental import pallas as pl
from jax.experimental.pallas import tpu as pltpu
from jax.experimental.pallas import tpu_sc as plsc

_N = 10000
_NP = 10112
_E = 320000
_F = 128
_NB = 8
_RH = 64
_OUT_S = 64
_OUT_V = 32
_MAX_R = 5.0
_NUM_NEIGH = 32.0
_INV_SQRT = 1.0 / float(_NUM_NEIGH) ** 0.5

_NC, _NS = 2, 16
_NW = _NC * _NS
_EW = _E // _NW
_CH = 128
_NCHUNKS = _E // _CH
_CHW = _NCHUNKS // _NW
_CH_REM = _NCHUNKS - _CHW * _NW
_ROWS_T = _NP // _NS
_WALL = 192

_BLK_E = 2560
_BLK_N = 2000
_GCH = 1000


def _sc_mesh():
    return plsc.VectorSubcoreMesh(core_axis_name="c", subcore_axis_name="s",
                                  num_cores=_NC, num_subcores=_NS)


def _prep_kernel(attr_ref, Wv_ref, xv_ref):
    xv_ref[...] = jnp.dot(attr_ref[...], Wv_ref[...],
                          preferred_element_type=jnp.float32)


def _run_prep(node_attrs, Wv):
    return pl.pallas_call(
        _prep_kernel,
        grid=(_N // _BLK_N,),
        in_specs=[
            pl.BlockSpec((_BLK_N, _F), lambda i: (i, 0)),
            pl.BlockSpec((_F, _OUT_V), lambda i: (0, 0)),
        ],
        out_specs=pl.BlockSpec((_BLK_N, _OUT_V), lambda i: (i, 0)),
        out_shape=jax.ShapeDtypeStruct((_N, _OUT_V), jnp.float32),
    )(node_attrs, Wv)


def _geo_body(p16_hbm, send_hbm, recv_hbm, d_hbm,
              idx_v, idxb_v, ps_v, pr_v, sem1, sem2):
    c = lax.axis_index("c")
    s = lax.axis_index("s")
    wid = s * _NC + c

    def chunk(i, carry):
        base = wid * _EW + i * _GCH
        pltpu.sync_copy(send_hbm.at[pl.ds(base, _GCH)], idx_v)
        g1 = pltpu.async_copy(p16_hbm.at[idx_v], ps_v, sem1)
        pltpu.sync_copy(recv_hbm.at[pl.ds(base, _GCH)], idxb_v)
        g2 = pltpu.async_copy(p16_hbm.at[idxb_v], pr_v, sem2)
        g1.wait()
        g2.wait()

        def grp(g, _):
            for j in range(8):
                e = g * 8 + j
                pr_v[e, :] = pr_v[e, :] - ps_v[e, :]
            return 0

        lax.fori_loop(0, _GCH // 8, grp, 0)
        pltpu.sync_copy(pr_v, d_hbm.at[pl.ds(base, _GCH)])
        return carry

    lax.fori_loop(0, _EW // _GCH, chunk, 0)


def _run_geo(p16, sender, receiver):
    f = pl.kernel(
        _geo_body,
        out_type=jax.ShapeDtypeStruct((_E, 16), jnp.float32),
        mesh=_sc_mesh(),
        compiler_params=pltpu.CompilerParams(use_tc_tiling_on_sc=False),
        scratch_types=[
            pltpu.VMEM((_GCH,), jnp.int32),
            pltpu.VMEM((_GCH,), jnp.int32),
            pltpu.VMEM((_GCH, 16), jnp.float32),
            pltpu.VMEM((_GCH, 16), jnp.float32),
            pltpu.SemaphoreType.DMA,
            pltpu.SemaphoreType.DMA,
        ],
    )
    return f(p16, sender, receiver)


def _edge_kernel(d_ref, W1_ref, b1_ref, W2s_ref, W2v_ref, ws_ref, wvn_ref):
    d = d_ref[:, 0:3]
    lengths = jnp.sqrt(jnp.sum(d * d, axis=1, keepdims=True))
    vec_n = d / (lengths + 1e-9)
    centers = lax.broadcasted_iota(jnp.int32, (1, _NB), 1).astype(jnp.float32) \
        * (_MAX_R / (_NB - 1))
    width = _MAX_R / _NB
    diff = (lengths - centers) * (1.0 / width)
    basis = jnp.exp(-(diff * diff))
    u = jnp.clip(1.0 - lengths * (1.0 / _MAX_R), 0.0, 1.0)
    env = u * u * (3.0 - 2.0 * u)
    pre = jnp.dot(basis, W1_ref[...], preferred_element_type=jnp.float32) + b1_ref[...]
    h = pre * jax.nn.sigmoid(pre)
    ws_ref[...] = jnp.dot(h, W2s_ref[...],
                          preferred_element_type=jnp.float32) * env
    wvn_ref[:, 0:_OUT_V] = jnp.dot(h, W2v_ref[...],
                                   preferred_element_type=jnp.float32) * env
    wvn_ref[:, _OUT_V:_OUT_V + 3] = vec_n
    wvn_ref[:, _OUT_V + 3:_F] = jnp.zeros((_BLK_E, _F - _OUT_V - 3), jnp.float32)


def _run_edge_dense(d, W1, b1, W2s, W2v):
    eb = lambda i: (i, 0)
    wb = lambda i: (0, 0)
    return pl.pallas_call(
        _edge_kernel,
        grid=(_E // _BLK_E,),
        in_specs=[
            pl.BlockSpec((_BLK_E, 16), eb),
            pl.BlockSpec((_NB, _RH), wb),
            pl.BlockSpec((1, _RH), wb),
            pl.BlockSpec((_RH, _F), wb),
            pl.BlockSpec((_RH, _OUT_V), wb),
        ],
        out_specs=[
            pl.BlockSpec((_BLK_E, _F), eb),
            pl.BlockSpec((_BLK_E, _F), eb),
        ],
        out_shape=[
            jax.ShapeDtypeStruct((_E, _F), jnp.float32),
            jax.ShapeDtypeStruct((_E, _F), jnp.float32),
        ],
    )(d, W1, b1.reshape(1, _RH), W2s, W2v)


def _scat_s_body(attr_hbm, wall_hbm, send_hbm, recv_hbm, zero_hbm,
                 out_hbm,
                 idxs_v, idxr_v, a_v, w_v, accum, sem, sem2, sem3):
    c = lax.axis_index("c")
    s = lax.axis_index("s")
    wid = s * _NC + c
    row0 = s * _ROWS_T
    for t in range(_ROWS_T // 64):
        pltpu.sync_copy(zero_hbm, accum.at[pl.ds(row0 + t * 64, 64)])
    _ZT = _ROWS_T - (_ROWS_T // 64) * 64
    if _ZT:
        pltpu.sync_copy(zero_hbm.at[pl.ds(0, _ZT)],
                        accum.at[pl.ds(row0 + (_ROWS_T // 64) * 64, _ZT)])
    plsc.subcore_barrier()
    nch = jnp.where(wid < _CH_REM, _CHW + 1, _CHW)

    def chunk1(i, carry):
        base = (wid + i * _NW) * _CH
        pltpu.sync_copy(send_hbm.at[pl.ds(base, _CH)], idxs_v)
        gat = pltpu.async_copy(attr_hbm.at[idxs_v], a_v, sem)
        lw = pltpu.async_copy(wall_hbm.at[pl.ds(base, _CH)], w_v, sem2)
        li = pltpu.async_copy(recv_hbm.at[pl.ds(base, _CH)], idxr_v, sem3)
        gat.wait()
        lw.wait()
        li.wait()

        def grpbody(g, _):
            for j in range(8):
                e = g * 8 + j
                for k in range(_F // 16):
                    sl = pl.ds(k * 16, 16)
                    a_v[e, sl] = a_v[e, sl] * w_v[e, sl]
            return 0

        lax.fori_loop(0, _CH // 8, grpbody, 0)
        pltpu.sync_copy(a_v, accum.at[idxr_v], add=True)
        return carry

    lax.fori_loop(0, nch, chunk1, 0)
    plsc.subcore_barrier()
    pltpu.sync_copy(accum.at[pl.ds(row0, _ROWS_T)],
                    out_hbm.at[c, pl.ds(row0, _ROWS_T)])


def _scat_v_body(xv_hbm, wall_hbm, epack_hbm,
                 out_hbm,
                 idxs_v, idxr_v, idx2_v, x_v, wv_v, m_v, accum,
                 sem, sem2, sem3):
    c = lax.axis_index("c")
    s = lax.axis_index("s")
    wid = s * _NC + c
    row0 = s * _ROWS_T

    def zrow(e, _):
        for k in range(_F // 16):
            m_v[e, pl.ds(k * 16, 16)] = jnp.zeros((16,), jnp.float32)
        return 0

    lax.fori_loop(0, _CH, zrow, 0)
    for t in range(_ROWS_T // _CH):
        pltpu.sync_copy(m_v, accum.at[pl.ds(row0 + t * _CH, _CH)])
    _TAIL = _ROWS_T - (_ROWS_T // _CH) * _CH
    if _TAIL:
        pltpu.sync_copy(m_v.at[pl.ds(0, _TAIL)],
                        accum.at[pl.ds(row0 + (_ROWS_T // _CH) * _CH, _TAIL)])
    plsc.subcore_barrier()
    nch = jnp.where(wid < _CH_REM, _CHW + 1, _CHW)

    def chunk2(i, carry):
        base = (wid + i * _NW) * _CH
        pltpu.sync_copy(epack_hbm.at[pl.ds(base, _CH)], idxs_v)

        def shft(g, _):
            sl = pl.ds(g * 16, 16)
            pk = idxs_v[sl]
            idx2_v[sl] = lax.shift_right_logical(pk & 0xFFFF, 2)
            idxr_v[sl] = lax.shift_right_logical(pk, 16)
            return 0

        lax.fori_loop(0, _CH // 16, shft, 0)
        gat = pltpu.async_copy(xv_hbm.at[idx2_v], x_v, sem)
        lw = pltpu.async_copy(wall_hbm.at[pl.ds(base, _CH)], wv_v, sem2)
        gat.wait()
        lw.wait()

        def grpbody(g, _):
            idxv = idxs_v[pl.ds(g * 16, 16)]
            for j in range(16):
                e = g * 16 + j
                off = (idxv[j] & 3) * _OUT_V
                x0 = x_v[e, pl.ds(off, 16)] * wv_v[e, pl.ds(0, 16)]
                x1 = x_v[e, pl.ds(off + 16, 16)] * wv_v[e, pl.ds(16, 16)]
                vn = wv_v[e, pl.ds(_OUT_V, 16)]
                vn0 = vn[0]
                vn1 = vn[1]
                vn2 = vn[2]
                m_v[e, pl.ds(0, 16)] = x0 * vn0
                m_v[e, pl.ds(16, 16)] = x1 * vn0
                m_v[e, pl.ds(32, 16)] = x0 * vn1
                m_v[e, pl.ds(48, 16)] = x1 * vn1
                m_v[e, pl.ds(64, 16)] = x0 * vn2
                m_v[e, pl.ds(80, 16)] = x1 * vn2
            return 0

        lax.fori_loop(0, _CH // 16, grpbody, 0)
        pltpu.sync_copy(m_v, accum.at[idxr_v], add=True)
        return carry

    lax.fori_loop(0, nch, chunk2, 0)
    plsc.subcore_barrier()
    pltpu.sync_copy(accum.at[pl.ds(row0, _ROWS_T)],
                    out_hbm.at[c, pl.ds(row0, _ROWS_T)])


def _run_scat_s(node_attrs, wall, sender, receiver, zero_s):
    f = pl.kernel(
        _scat_s_body,
        out_type=jax.ShapeDtypeStruct((_NC, _NP, _F), jnp.float32),
        mesh=_sc_mesh(),
        compiler_params=pltpu.CompilerParams(use_tc_tiling_on_sc=True),
        scratch_types=[
            pltpu.VMEM((_CH,), jnp.int32),
            pltpu.VMEM((_CH,), jnp.int32),
            pltpu.VMEM((_CH, _F), jnp.float32),
            pltpu.VMEM((_CH, _F), jnp.float32),
            pltpu.VMEM_SHARED((_NP, _F), jnp.float32),
            pltpu.SemaphoreType.DMA,
            pltpu.SemaphoreType.DMA,
            pltpu.SemaphoreType.DMA,
        ],
    )
    return f(node_attrs, wall, sender, receiver, zero_s)


def _run_scat_v(xvp, wall, epack):
    f = pl.kernel(
        _scat_v_body,
        out_type=jax.ShapeDtypeStruct((_NC, _NP, _F), jnp.float32),
        mesh=_sc_mesh(),
        compiler_params=pltpu.CompilerParams(use_tc_tiling_on_sc=True),
        scratch_types=[
            pltpu.VMEM((_CH,), jnp.int32),
            pltpu.VMEM((_CH,), jnp.int32),
            pltpu.VMEM((_CH,), jnp.int32),
            pltpu.VMEM((_CH, _F), jnp.float32),
            pltpu.VMEM((_CH, _F), jnp.float32),
            pltpu.VMEM((_CH, _F), jnp.float32),
            pltpu.VMEM_SHARED((_NP, _F), jnp.float32),
            pltpu.SemaphoreType.DMA,
            pltpu.SemaphoreType.DMA,
            pltpu.SemaphoreType.DMA,
        ],
    )
    return f(xvp, wall, epack)


def _final_kernel(s0_ref, s1_ref, v0_ref, v1_ref, Ws_ref, outs_ref, aggv_ref):
    agg_s = (s0_ref[0] + s1_ref[0]) * _INV_SQRT
    outs_ref[...] = jnp.dot(agg_s, Ws_ref[...],
                            preferred_element_type=jnp.float32)
    v = (v0_ref[0] + v1_ref[0]) * _INV_SQRT
    aggv_ref[...] = v[:, 0:3 * _OUT_V]


def _run_final(s_part, v_part, Ws):
    return pl.pallas_call(
        _final_kernel,
        grid=(_N // _BLK_N,),
        in_specs=[
            pl.BlockSpec((1, _BLK_N, _F), lambda i: (0, i, 0)),
            pl.BlockSpec((1, _BLK_N, _F), lambda i: (1, i, 0)),
            pl.BlockSpec((1, _BLK_N, _F), lambda i: (0, i, 0)),
            pl.BlockSpec((1, _BLK_N, _F), lambda i: (1, i, 0)),
            pl.BlockSpec((_F, _OUT_S), lambda i: (0, 0)),
        ],
        out_specs=[
            pl.BlockSpec((_BLK_N, _OUT_S), lambda i: (i, 0)),
            pl.BlockSpec((_BLK_N, 3 * _OUT_V), lambda i: (i, 0)),
        ],
        out_shape=[
            jax.ShapeDtypeStruct((_N, _OUT_S), jnp.float32),
            jax.ShapeDtypeStruct((_N, 3 * _OUT_V), jnp.float32),
        ],
    )(s_part, s_part, v_part, v_part, Ws)


def kernel(node_attrs, positions, edge_index, shifts, W1, b1, W2s, W2v, Wv, Ws):
    sender = edge_index[0]
    receiver = edge_index[1]
    zero_s = jnp.zeros((64, _F), jnp.float32)
    epack = sender | (receiver << 16)
    p16 = jnp.pad(positions, ((0, 0), (0, 13)))

    xv = _run_prep(node_attrs, Wv)
    xvp = jnp.pad(xv, ((0, _NP - _N), (0, 0))).reshape(_NP // 4, 4 * _OUT_V)
    d = _run_geo(p16, sender, receiver)
    ws, wvn = _run_edge_dense(d, W1, b1, W2s, W2v)
    s_part = _run_scat_s(node_attrs, ws, sender, receiver, zero_s)
    v_part = _run_scat_v(xvp, wvn, epack)
    out_s, aggv_km = _run_final(s_part, v_part, Ws)

    agg_v = aggv_km.reshape(_N, 3, _OUT_V).transpose(0, 2, 1).reshape(_N, 3 * _OUT_V)
    return jnp.concatenate([out_s, agg_v], axis=1)


# speedup vs baseline: 32.9076x; 1.0481x over previous
import functools

import jax
import jax.numpy as jnp
from jax import lax
from jax.experimental import pallas as pl
from jax.experimental.pallas import tpu as pltpu
from jax.experimental.pallas import tpu_sc as plsc

_N = 10000
_NP = 10112
_E = 320000
_F = 128
_NB = 8
_RH = 64
_OUT_S = 64
_OUT_V = 32
_MAX_R = 5.0
_NUM_NEIGH = 32.0
_INV_SQRT = 1.0 / float(_NUM_NEIGH) ** 0.5

_NC, _NS = 2, 16
_NW = _NC * _NS
_EW = _E // _NW
_CH = 128
_NCHUNKS = _E // _CH
_CHW = _NCHUNKS // _NW
_CH_REM = _NCHUNKS - _CHW * _NW
_ROWS_T = _NP // _NS
_WALL = 192

_BLK_E = 2560
_BLK_N = 2000
_GCH = 1000


def _sc_mesh():
    return plsc.VectorSubcoreMesh(core_axis_name="c", subcore_axis_name="s",
                                  num_cores=_NC, num_subcores=_NS)


def _prep_kernel(attr_ref, Wv_ref, xv_ref):
    xv_ref[...] = jnp.dot(attr_ref[...], Wv_ref[...],
                          preferred_element_type=jnp.float32)


def _run_prep(node_attrs, Wv):
    return pl.pallas_call(
        _prep_kernel,
        grid=(_N // _BLK_N,),
        in_specs=[
            pl.BlockSpec((_BLK_N, _F), lambda i: (i, 0)),
            pl.BlockSpec((_F, _OUT_V), lambda i: (0, 0)),
        ],
        out_specs=pl.BlockSpec((_BLK_N, _OUT_V), lambda i: (i, 0)),
        out_shape=jax.ShapeDtypeStruct((_N, _OUT_V), jnp.float32),
    )(node_attrs, Wv)


def _geo_body(p16_hbm, send_hbm, recv_hbm, d_hbm,
              idx_v, idxb_v, ps_v, pr_v, sem1, sem2):
    c = lax.axis_index("c")
    s = lax.axis_index("s")
    wid = s * _NC + c

    def chunk(i, carry):
        base = wid * _EW + i * _GCH
        pltpu.sync_copy(send_hbm.at[pl.ds(base, _GCH)], idx_v)
        g1 = pltpu.async_copy(p16_hbm.at[idx_v], ps_v, sem1)
        pltpu.sync_copy(recv_hbm.at[pl.ds(base, _GCH)], idxb_v)
        g2 = pltpu.async_copy(p16_hbm.at[idxb_v], pr_v, sem2)
        g1.wait()
        g2.wait()

        def grp(g, _):
            for j in range(8):
                e = g * 8 + j
                pr_v[e, :] = pr_v[e, :] - ps_v[e, :]
            return 0

        lax.fori_loop(0, _GCH // 8, grp, 0)
        pltpu.sync_copy(pr_v, d_hbm.at[pl.ds(base, _GCH)])
        return carry

    lax.fori_loop(0, _EW // _GCH, chunk, 0)


def _run_geo(p16, sender, receiver):
    f = pl.kernel(
        _geo_body,
        out_type=jax.ShapeDtypeStruct((_E, 16), jnp.float32),
        mesh=_sc_mesh(),
        compiler_params=pltpu.CompilerParams(use_tc_tiling_on_sc=False),
        scratch_types=[
            pltpu.VMEM((_GCH,), jnp.int32),
            pltpu.VMEM((_GCH,), jnp.int32),
            pltpu.VMEM((_GCH, 16), jnp.float32),
            pltpu.VMEM((_GCH, 16), jnp.float32),
            pltpu.SemaphoreType.DMA,
            pltpu.SemaphoreType.DMA,
        ],
    )
    return f(p16, sender, receiver)


def _edge_kernel(d_ref, W1_ref, b1_ref, W2s_ref, W2v_ref, ws_ref, wvn_ref):
    d = d_ref[:, 0:3]
    lengths = jnp.sqrt(jnp.sum(d * d, axis=1, keepdims=True))
    vec_n = d / (lengths + 1e-9)
    centers = lax.broadcasted_iota(jnp.int32, (1, _NB), 1).astype(jnp.float32) \
        * (_MAX_R / (_NB - 1))
    width = _MAX_R / _NB
    diff = (lengths - centers) * (1.0 / width)
    basis = jnp.exp(-(diff * diff))
    u = jnp.clip(1.0 - lengths * (1.0 / _MAX_R), 0.0, 1.0)
    env = u * u * (3.0 - 2.0 * u)
    pre = jnp.dot(basis, W1_ref[...], preferred_element_type=jnp.float32) + b1_ref[...]
    h = pre * jax.nn.sigmoid(pre)
    ws_ref[...] = jnp.dot(h, W2s_ref[...],
                          preferred_element_type=jnp.float32) * env
    wvn_ref[:, 0:_OUT_V] = jnp.dot(h, W2v_ref[...],
                                   preferred_element_type=jnp.float32) * env
    wvn_ref[:, _OUT_V:_OUT_V + 3] = vec_n


def _run_edge_dense(d, W1, b1, W2s, W2v):
    eb = lambda i: (i, 0)
    wb = lambda i: (0, 0)
    return pl.pallas_call(
        _edge_kernel,
        grid=(_E // _BLK_E,),
        in_specs=[
            pl.BlockSpec((_BLK_E, 16), eb),
            pl.BlockSpec((_NB, _RH), wb),
            pl.BlockSpec((1, _RH), wb),
            pl.BlockSpec((_RH, _F), wb),
            pl.BlockSpec((_RH, _OUT_V), wb),
        ],
        out_specs=[
            pl.BlockSpec((_BLK_E, _F), eb),
            pl.BlockSpec((_BLK_E, _F), eb),
        ],
        out_shape=[
            jax.ShapeDtypeStruct((_E, _F), jnp.float32),
            jax.ShapeDtypeStruct((_E, _F), jnp.float32),
        ],
    )(d, W1, b1.reshape(1, _RH), W2s, W2v)


def _scat_s_body(attr_hbm, wall_hbm, send_hbm, recv_hbm, zero_hbm,
                 out_hbm,
                 idxs_v, idxr_v, idxr2_v, a_v, w_v, prod_v, accum,
                 sem, sem2, sem3, sem4):
    c = lax.axis_index("c")
    s = lax.axis_index("s")
    wid = s * _NC + c
    row0 = s * _ROWS_T
    for t in range(_ROWS_T // 64):
        pltpu.sync_copy(zero_hbm, accum.at[pl.ds(row0 + t * 64, 64)])
    _ZT = _ROWS_T - (_ROWS_T // 64) * 64
    if _ZT:
        pltpu.sync_copy(zero_hbm.at[pl.ds(0, _ZT)],
                        accum.at[pl.ds(row0 + (_ROWS_T // 64) * 64, _ZT)])
    plsc.subcore_barrier()
    nch = jnp.where(wid < _CH_REM, _CHW + 1, _CHW)

    def chunk1(i, carry):
        base = (wid + i * _NW) * _CH
        pltpu.sync_copy(send_hbm.at[pl.ds(base, _CH)], idxs_v)
        gat = pltpu.async_copy(attr_hbm.at[idxs_v], a_v, sem)
        lw = pltpu.async_copy(wall_hbm.at[pl.ds(base, _CH)], w_v, sem2)
        li = pltpu.async_copy(recv_hbm.at[pl.ds(base, _CH)], idxr2_v, sem3)
        gat.wait()
        lw.wait()

        @pl.when(i > 0)
        def _():
            pltpu.make_async_copy(prod_v, accum.at[idxr_v], sem4).wait()

        def grpbody(g, _):
            for j in range(8):
                e = g * 8 + j
                for k in range(_F // 16):
                    sl = pl.ds(k * 16, 16)
                    prod_v[e, sl] = a_v[e, sl] * w_v[e, sl]
            return 0

        lax.fori_loop(0, _CH // 8, grpbody, 0)
        li.wait()

        def cpidx(g, _):
            sl = pl.ds(g * 16, 16)
            idxr_v[sl] = idxr2_v[sl]
            return 0

        lax.fori_loop(0, _CH // 16, cpidx, 0)
        pltpu.async_copy(prod_v, accum.at[idxr_v], sem4, add=True)
        return carry

    lax.fori_loop(0, nch, chunk1, 0)
    pltpu.make_async_copy(prod_v, accum.at[idxr_v], sem4).wait()
    plsc.subcore_barrier()
    pltpu.sync_copy(accum.at[pl.ds(row0, _ROWS_T)],
                    out_hbm.at[c, pl.ds(row0, _ROWS_T)])


def _scat_v_body(xv_hbm, wall_hbm, epack_hbm,
                 out_hbm,
                 idxs_v, idxr_v, idx2_v, x_v, wv_v, m_v, accum,
                 sem, sem2, sem3):
    c = lax.axis_index("c")
    s = lax.axis_index("s")
    wid = s * _NC + c
    row0 = s * _ROWS_T

    def zrow(e, _):
        for k in range(_F // 16):
            m_v[e, pl.ds(k * 16, 16)] = jnp.zeros((16,), jnp.float32)
        return 0

    lax.fori_loop(0, _CH, zrow, 0)
    for t in range(_ROWS_T // _CH):
        pltpu.sync_copy(m_v, accum.at[pl.ds(row0 + t * _CH, _CH)])
    _TAIL = _ROWS_T - (_ROWS_T // _CH) * _CH
    if _TAIL:
        pltpu.sync_copy(m_v.at[pl.ds(0, _TAIL)],
                        accum.at[pl.ds(row0 + (_ROWS_T // _CH) * _CH, _TAIL)])
    plsc.subcore_barrier()
    nch = jnp.where(wid < _CH_REM, _CHW + 1, _CHW)

    def chunk2(i, carry):
        base = (wid + i * _NW) * _CH
        pltpu.sync_copy(epack_hbm.at[pl.ds(base, _CH)], idxs_v)

        def shft(g, _):
            sl = pl.ds(g * 16, 16)
            pk = idxs_v[sl]
            idx2_v[sl] = lax.shift_right_logical(pk & 0xFFFF, 2)
            idxr_v[sl] = lax.shift_right_logical(pk, 16)
            return 0

        lax.fori_loop(0, _CH // 16, shft, 0)
        gat = pltpu.async_copy(xv_hbm.at[idx2_v], x_v, sem)
        lw = pltpu.async_copy(wall_hbm.at[pl.ds(base, _CH)], wv_v, sem2)
        gat.wait()
        lw.wait()

        def grpbody(g, _):
            idxv = idxs_v[pl.ds(g * 16, 16)]
            for j in range(16):
                e = g * 16 + j
                off = (idxv[j] & 3) * _OUT_V
                x0 = x_v[e, pl.ds(off, 16)] * wv_v[e, pl.ds(0, 16)]
                x1 = x_v[e, pl.ds(off + 16, 16)] * wv_v[e, pl.ds(16, 16)]
                vn = wv_v[e, pl.ds(_OUT_V, 16)]
                vn0 = vn[0]
                vn1 = vn[1]
                vn2 = vn[2]
                m_v[e, pl.ds(0, 16)] = x0 * vn0
                m_v[e, pl.ds(16, 16)] = x1 * vn0
                m_v[e, pl.ds(32, 16)] = x0 * vn1
                m_v[e, pl.ds(48, 16)] = x1 * vn1
                m_v[e, pl.ds(64, 16)] = x0 * vn2
                m_v[e, pl.ds(80, 16)] = x1 * vn2
            return 0

        lax.fori_loop(0, _CH // 16, grpbody, 0)
        pltpu.sync_copy(m_v, accum.at[idxr_v], add=True)
        return carry

    lax.fori_loop(0, nch, chunk2, 0)
    plsc.subcore_barrier()
    pltpu.sync_copy(accum.at[pl.ds(row0, _ROWS_T)],
                    out_hbm.at[c, pl.ds(row0, _ROWS_T)])


def _run_scat_s(node_attrs, wall, sender, receiver, zero_s):
    f = pl.kernel(
        _scat_s_body,
        out_type=jax.ShapeDtypeStruct((_NC, _NP, _F), jnp.float32),
        mesh=_sc_mesh(),
        compiler_params=pltpu.CompilerParams(use_tc_tiling_on_sc=True),
        scratch_types=[
            pltpu.VMEM((_CH,), jnp.int32),
            pltpu.VMEM((_CH,), jnp.int32),
            pltpu.VMEM((_CH,), jnp.int32),
            pltpu.VMEM((_CH, _F), jnp.float32),
            pltpu.VMEM((_CH, _F), jnp.float32),
            pltpu.VMEM((_CH, _F), jnp.float32),
            pltpu.VMEM_SHARED((_NP, _F), jnp.float32),
            pltpu.SemaphoreType.DMA,
            pltpu.SemaphoreType.DMA,
            pltpu.SemaphoreType.DMA,
            pltpu.SemaphoreType.DMA,
        ],
    )
    return f(node_attrs, wall, sender, receiver, zero_s)


def _run_scat_v(xvp, wall, epack):
    f = pl.kernel(
        _scat_v_body,
        out_type=jax.ShapeDtypeStruct((_NC, _NP, _F), jnp.float32),
        mesh=_sc_mesh(),
        compiler_params=pltpu.CompilerParams(use_tc_tiling_on_sc=True),
        scratch_types=[
            pltpu.VMEM((_CH,), jnp.int32),
            pltpu.VMEM((_CH,), jnp.int32),
            pltpu.VMEM((_CH,), jnp.int32),
            pltpu.VMEM((_CH, _F), jnp.float32),
            pltpu.VMEM((_CH, _F), jnp.float32),
            pltpu.VMEM((_CH, _F), jnp.float32),
            pltpu.VMEM_SHARED((_NP, _F), jnp.float32),
            pltpu.SemaphoreType.DMA,
            pltpu.SemaphoreType.DMA,
            pltpu.SemaphoreType.DMA,
        ],
    )
    return f(xvp, wall, epack)


def _final_kernel(s0_ref, s1_ref, v0_ref, v1_ref, Ws_ref, outs_ref, aggv_ref):
    agg_s = (s0_ref[0] + s1_ref[0]) * _INV_SQRT
    outs_ref[...] = jnp.dot(agg_s, Ws_ref[...],
                            preferred_element_type=jnp.float32)
    v = (v0_ref[0] + v1_ref[0]) * _INV_SQRT
    aggv_ref[...] = v[:, 0:3 * _OUT_V]


def _run_final(s_part, v_part, Ws):
    return pl.pallas_call(
        _final_kernel,
        grid=(_N // _BLK_N,),
        in_specs=[
            pl.BlockSpec((1, _BLK_N, _F), lambda i: (0, i, 0)),
            pl.BlockSpec((1, _BLK_N, _F), lambda i: (1, i, 0)),
            pl.BlockSpec((1, _BLK_N, _F), lambda i: (0, i, 0)),
            pl.BlockSpec((1, _BLK_N, _F), lambda i: (1, i, 0)),
            pl.BlockSpec((_F, _OUT_S), lambda i: (0, 0)),
        ],
        out_specs=[
            pl.BlockSpec((_BLK_N, _OUT_S), lambda i: (i, 0)),
            pl.BlockSpec((_BLK_N, 3 * _OUT_V), lambda i: (i, 0)),
        ],
        out_shape=[
            jax.ShapeDtypeStruct((_N, _OUT_S), jnp.float32),
            jax.ShapeDtypeStruct((_N, 3 * _OUT_V), jnp.float32),
        ],
    )(s_part, s_part, v_part, v_part, Ws)


def kernel(node_attrs, positions, edge_index, shifts, W1, b1, W2s, W2v, Wv, Ws):
    sender = edge_index[0]
    receiver = edge_index[1]
    zero_s = jnp.zeros((64, _F), jnp.float32)
    epack = sender | (receiver << 16)
    p16 = jnp.pad(positions, ((0, 0), (0, 13)))

    xv = _run_prep(node_attrs, Wv)
    xvp = jnp.pad(xv, ((0, _NP - _N), (0, 0))).reshape(_NP // 4, 4 * _OUT_V)
    d = _run_geo(p16, sender, receiver)
    ws, wvn = _run_edge_dense(d, W1, b1, W2s, W2v)
    s_part = _run_scat_s(node_attrs, ws, sender, receiver, zero_s)
    v_part = _run_scat_v(xvp, wvn, epack)
    out_s, aggv_km = _run_final(s_part, v_part, Ws)

    agg_v = aggv_km.reshape(_N, 3, _OUT_V).transpose(0, 2, 1).reshape(_N, 3 * _OUT_V)
    return jnp.concatenate([out_s, agg_v], axis=1)


# speedup vs baseline: 34.3252x; 1.0431x over previous
import functools

import jax
import jax.numpy as jnp
from jax import lax
from jax.experimental import pallas as pl
from jax.experimental.pallas import tpu as pltpu
from jax.experimental.pallas import tpu_sc as plsc

_N = 10000
_NP = 10112
_E = 320000
_F = 128
_NB = 8
_RH = 64
_OUT_S = 64
_OUT_V = 32
_MAX_R = 5.0
_NUM_NEIGH = 32.0
_INV_SQRT = 1.0 / float(_NUM_NEIGH) ** 0.5

_NC, _NS = 2, 16
_NW = _NC * _NS
_EW = _E // _NW
_CH = 128
_NCHUNKS = _E // _CH
_CHW = _NCHUNKS // _NW
_CH_REM = _NCHUNKS - _CHW * _NW
_ROWS_T = _NP // _NS
_WALL = 192

_BLK_E = 2560
_BLK_N = 2000
_GCH = 1000


def _sc_mesh():
    return plsc.VectorSubcoreMesh(core_axis_name="c", subcore_axis_name="s",
                                  num_cores=_NC, num_subcores=_NS)


def _prep_kernel(attr_ref, Wv_ref, xv_ref):
    xv_ref[...] = jnp.dot(attr_ref[...], Wv_ref[...],
                          preferred_element_type=jnp.float32)


def _run_prep(node_attrs, Wv):
    return pl.pallas_call(
        _prep_kernel,
        grid=(_N // _BLK_N,),
        in_specs=[
            pl.BlockSpec((_BLK_N, _F), lambda i: (i, 0)),
            pl.BlockSpec((_F, _OUT_V), lambda i: (0, 0)),
        ],
        out_specs=pl.BlockSpec((_BLK_N, _OUT_V), lambda i: (i, 0)),
        out_shape=jax.ShapeDtypeStruct((_N, _OUT_V), jnp.float32),
    )(node_attrs, Wv)


def _geo_body(p16_hbm, send_hbm, recv_hbm, d_hbm,
              idx_v, idxb_v, ps_v, pr_v, sem1, sem2):
    c = lax.axis_index("c")
    s = lax.axis_index("s")
    wid = s * _NC + c

    def chunk(i, carry):
        base = wid * _EW + i * _GCH
        pltpu.sync_copy(send_hbm.at[pl.ds(base, _GCH)], idx_v)
        g1 = pltpu.async_copy(p16_hbm.at[idx_v], ps_v, sem1)
        pltpu.sync_copy(recv_hbm.at[pl.ds(base, _GCH)], idxb_v)
        g2 = pltpu.async_copy(p16_hbm.at[idxb_v], pr_v, sem2)
        g1.wait()
        g2.wait()

        def grp(g, _):
            for j in range(8):
                e = g * 8 + j
                pr_v[e, :] = pr_v[e, :] - ps_v[e, :]
            return 0

        lax.fori_loop(0, _GCH // 8, grp, 0)
        pltpu.sync_copy(pr_v, d_hbm.at[pl.ds(base, _GCH)])
        return carry

    lax.fori_loop(0, _EW // _GCH, chunk, 0)


def _run_geo(p16, sender, receiver):
    f = pl.kernel(
        _geo_body,
        out_type=jax.ShapeDtypeStruct((_E, 16), jnp.float32),
        mesh=_sc_mesh(),
        compiler_params=pltpu.CompilerParams(use_tc_tiling_on_sc=False),
        scratch_types=[
            pltpu.VMEM((_GCH,), jnp.int32),
            pltpu.VMEM((_GCH,), jnp.int32),
            pltpu.VMEM((_GCH, 16), jnp.float32),
            pltpu.VMEM((_GCH, 16), jnp.float32),
            pltpu.SemaphoreType.DMA,
            pltpu.SemaphoreType.DMA,
        ],
    )
    return f(p16, sender, receiver)


def _edge_kernel(d_ref, W1_ref, b1_ref, W2s_ref, W2v_ref, ws_ref, wvn_ref):
    d = d_ref[:, 0:3]
    lengths = jnp.sqrt(jnp.sum(d * d, axis=1, keepdims=True))
    vec_n = d / (lengths + 1e-9)
    centers = lax.broadcasted_iota(jnp.int32, (1, _NB), 1).astype(jnp.float32) \
        * (_MAX_R / (_NB - 1))
    width = _MAX_R / _NB
    diff = (lengths - centers) * (1.0 / width)
    basis = jnp.exp(-(diff * diff))
    u = jnp.clip(1.0 - lengths * (1.0 / _MAX_R), 0.0, 1.0)
    env = u * u * (3.0 - 2.0 * u)
    pre = jnp.dot(basis, W1_ref[...], preferred_element_type=jnp.float32) + b1_ref[...]
    h = pre * jax.nn.sigmoid(pre)
    ws_ref[...] = jnp.dot(h, W2s_ref[...],
                          preferred_element_type=jnp.float32) * env
    wvn_ref[:, 0:_OUT_V] = jnp.dot(h, W2v_ref[...],
                                   preferred_element_type=jnp.float32) * env
    wvn_ref[:, _OUT_V:_OUT_V + 3] = vec_n


def _run_edge_dense(d, W1, b1, W2s, W2v):
    eb = lambda i: (i, 0)
    wb = lambda i: (0, 0)
    return pl.pallas_call(
        _edge_kernel,
        grid=(_E // _BLK_E,),
        in_specs=[
            pl.BlockSpec((_BLK_E, 16), eb),
            pl.BlockSpec((_NB, _RH), wb),
            pl.BlockSpec((1, _RH), wb),
            pl.BlockSpec((_RH, _F), wb),
            pl.BlockSpec((_RH, _OUT_V), wb),
        ],
        out_specs=[
            pl.BlockSpec((_BLK_E, _F), eb),
            pl.BlockSpec((_BLK_E, _F), eb),
        ],
        out_shape=[
            jax.ShapeDtypeStruct((_E, _F), jnp.float32),
            jax.ShapeDtypeStruct((_E, _F), jnp.float32),
        ],
    )(d, W1, b1.reshape(1, _RH), W2s, W2v)


def _scat_s_body(attr_hbm, wall_hbm, send_hbm, recv_hbm, zero_hbm,
                 out_hbm,
                 idxs_v, idxr_v, idxr2_v, a_v, w_v, prod_v, accum,
                 sem, sem2, sem3, sem4):
    c = lax.axis_index("c")
    s = lax.axis_index("s")
    wid = s * _NC + c
    row0 = s * _ROWS_T
    for t in range(_ROWS_T // 64):
        pltpu.sync_copy(zero_hbm, accum.at[pl.ds(row0 + t * 64, 64)])
    _ZT = _ROWS_T - (_ROWS_T // 64) * 64
    if _ZT:
        pltpu.sync_copy(zero_hbm.at[pl.ds(0, _ZT)],
                        accum.at[pl.ds(row0 + (_ROWS_T // 64) * 64, _ZT)])
    plsc.subcore_barrier()
    nch = jnp.where(wid < _CH_REM, _CHW + 1, _CHW)

    def chunk1(i, carry):
        base = (wid + i * _NW) * _CH
        pltpu.sync_copy(send_hbm.at[pl.ds(base, _CH)], idxs_v)
        gat = pltpu.async_copy(attr_hbm.at[idxs_v], a_v, sem)
        lw = pltpu.async_copy(wall_hbm.at[pl.ds(base, _CH)], w_v, sem2)
        li = pltpu.async_copy(recv_hbm.at[pl.ds(base, _CH)], idxr2_v, sem3)
        gat.wait()
        lw.wait()

        @pl.when(i > 0)
        def _():
            pltpu.make_async_copy(prod_v, accum.at[idxr_v], sem4).wait()

        def grpbody(g, _):
            for j in range(8):
                e = g * 8 + j
                for k in range(_F // 16):
                    sl = pl.ds(k * 16, 16)
                    prod_v[e, sl] = a_v[e, sl] * w_v[e, sl]
            return 0

        lax.fori_loop(0, _CH // 8, grpbody, 0)
        li.wait()

        def cpidx(g, _):
            sl = pl.ds(g * 16, 16)
            idxr_v[sl] = idxr2_v[sl]
            return 0

        lax.fori_loop(0, _CH // 16, cpidx, 0)
        pltpu.async_copy(prod_v, accum.at[idxr_v], sem4, add=True)
        return carry

    lax.fori_loop(0, nch, chunk1, 0)
    pltpu.make_async_copy(prod_v, accum.at[idxr_v], sem4).wait()
    plsc.subcore_barrier()
    pltpu.sync_copy(accum.at[pl.ds(row0, _ROWS_T)],
                    out_hbm.at[c, pl.ds(row0, _ROWS_T)])


def _scat_v_body(xv_hbm, wall_hbm, epack_hbm,
                 out_hbm,
                 idxs_v, idxr_v, idxr2_v, idx2_v, x_v, wv_v, m_v, accum,
                 sem, sem2, sem3):
    c = lax.axis_index("c")
    s = lax.axis_index("s")
    wid = s * _NC + c
    row0 = s * _ROWS_T

    def zrow(e, _):
        for k in range(_F // 16):
            m_v[e, pl.ds(k * 16, 16)] = jnp.zeros((16,), jnp.float32)
        return 0

    lax.fori_loop(0, _CH, zrow, 0)
    for t in range(_ROWS_T // _CH):
        pltpu.sync_copy(m_v, accum.at[pl.ds(row0 + t * _CH, _CH)])
    _TAIL = _ROWS_T - (_ROWS_T // _CH) * _CH
    if _TAIL:
        pltpu.sync_copy(m_v.at[pl.ds(0, _TAIL)],
                        accum.at[pl.ds(row0 + (_ROWS_T // _CH) * _CH, _TAIL)])
    plsc.subcore_barrier()
    nch = jnp.where(wid < _CH_REM, _CHW + 1, _CHW)

    def chunk2(i, carry):
        base = (wid + i * _NW) * _CH
        pltpu.sync_copy(epack_hbm.at[pl.ds(base, _CH)], idxs_v)

        def shft(g, _):
            sl = pl.ds(g * 16, 16)
            pk = idxs_v[sl]
            idx2_v[sl] = lax.shift_right_logical(pk & 0xFFFF, 2)
            idxr2_v[sl] = lax.shift_right_logical(pk, 16)
            return 0

        lax.fori_loop(0, _CH // 16, shft, 0)
        gat = pltpu.async_copy(xv_hbm.at[idx2_v], x_v, sem)
        lw = pltpu.async_copy(wall_hbm.at[pl.ds(base, _CH)], wv_v, sem2)
        gat.wait()
        lw.wait()

        @pl.when(i > 0)
        def _():
            pltpu.make_async_copy(m_v, accum.at[idxr_v], sem3).wait()

        def grpbody(g, _):
            idxv = idxs_v[pl.ds(g * 16, 16)]
            for j in range(16):
                e = g * 16 + j
                off = (idxv[j] & 3) * _OUT_V
                x0 = x_v[e, pl.ds(off, 16)] * wv_v[e, pl.ds(0, 16)]
                x1 = x_v[e, pl.ds(off + 16, 16)] * wv_v[e, pl.ds(16, 16)]
                vn = wv_v[e, pl.ds(_OUT_V, 16)]
                vn0 = vn[0]
                vn1 = vn[1]
                vn2 = vn[2]
                m_v[e, pl.ds(0, 16)] = x0 * vn0
                m_v[e, pl.ds(16, 16)] = x1 * vn0
                m_v[e, pl.ds(32, 16)] = x0 * vn1
                m_v[e, pl.ds(48, 16)] = x1 * vn1
                m_v[e, pl.ds(64, 16)] = x0 * vn2
                m_v[e, pl.ds(80, 16)] = x1 * vn2
            return 0

        lax.fori_loop(0, _CH // 16, grpbody, 0)

        def cpidx(g, _):
            sl = pl.ds(g * 16, 16)
            idxr_v[sl] = idxr2_v[sl]
            return 0

        lax.fori_loop(0, _CH // 16, cpidx, 0)
        pltpu.async_copy(m_v, accum.at[idxr_v], sem3, add=True)
        return carry

    lax.fori_loop(0, nch, chunk2, 0)
    pltpu.make_async_copy(m_v, accum.at[idxr_v], sem3).wait()
    plsc.subcore_barrier()
    pltpu.sync_copy(accum.at[pl.ds(row0, _ROWS_T)],
                    out_hbm.at[c, pl.ds(row0, _ROWS_T)])


def _run_scat_s(node_attrs, wall, sender, receiver, zero_s):
    f = pl.kernel(
        _scat_s_body,
        out_type=jax.ShapeDtypeStruct((_NC, _NP, _F), jnp.float32),
        mesh=_sc_mesh(),
        compiler_params=pltpu.CompilerParams(use_tc_tiling_on_sc=True),
        scratch_types=[
            pltpu.VMEM((_CH,), jnp.int32),
            pltpu.VMEM((_CH,), jnp.int32),
            pltpu.VMEM((_CH,), jnp.int32),
            pltpu.VMEM((_CH, _F), jnp.float32),
            pltpu.VMEM((_CH, _F), jnp.float32),
            pltpu.VMEM((_CH, _F), jnp.float32),
            pltpu.VMEM_SHARED((_NP, _F), jnp.float32),
            pltpu.SemaphoreType.DMA,
            pltpu.SemaphoreType.DMA,
            pltpu.SemaphoreType.DMA,
            pltpu.SemaphoreType.DMA,
        ],
    )
    return f(node_attrs, wall, sender, receiver, zero_s)


def _run_scat_v(xvp, wall, epack):
    f = pl.kernel(
        _scat_v_body,
        out_type=jax.ShapeDtypeStruct((_NC, _NP, _F), jnp.float32),
        mesh=_sc_mesh(),
        compiler_params=pltpu.CompilerParams(use_tc_tiling_on_sc=True),
        scratch_types=[
            pltpu.VMEM((_CH,), jnp.int32),
            pltpu.VMEM((_CH,), jnp.int32),
            pltpu.VMEM((_CH,), jnp.int32),
            pltpu.VMEM((_CH,), jnp.int32),
            pltpu.VMEM((_CH, _F), jnp.float32),
            pltpu.VMEM((_CH, _F), jnp.float32),
            pltpu.VMEM((_CH, _F), jnp.float32),
            pltpu.VMEM_SHARED((_NP, _F), jnp.float32),
            pltpu.SemaphoreType.DMA,
            pltpu.SemaphoreType.DMA,
            pltpu.SemaphoreType.DMA,
        ],
    )
    return f(xvp, wall, epack)


def _final_kernel(s0_ref, s1_ref, v0_ref, v1_ref, Ws_ref, outs_ref, aggv_ref):
    agg_s = (s0_ref[0] + s1_ref[0]) * _INV_SQRT
    outs_ref[...] = jnp.dot(agg_s, Ws_ref[...],
                            preferred_element_type=jnp.float32)
    v = (v0_ref[0] + v1_ref[0]) * _INV_SQRT
    aggv_ref[...] = v[:, 0:3 * _OUT_V]


def _run_final(s_part, v_part, Ws):
    return pl.pallas_call(
        _final_kernel,
        grid=(_N // _BLK_N,),
        in_specs=[
            pl.BlockSpec((1, _BLK_N, _F), lambda i: (0, i, 0)),
            pl.BlockSpec((1, _BLK_N, _F), lambda i: (1, i, 0)),
            pl.BlockSpec((1, _BLK_N, _F), lambda i: (0, i, 0)),
            pl.BlockSpec((1, _BLK_N, _F), lambda i: (1, i, 0)),
            pl.BlockSpec((_F, _OUT_S), lambda i: (0, 0)),
        ],
        out_specs=[
            pl.BlockSpec((_BLK_N, _OUT_S), lambda i: (i, 0)),
            pl.BlockSpec((_BLK_N, 3 * _OUT_V), lambda i: (i, 0)),
        ],
        out_shape=[
            jax.ShapeDtypeStruct((_N, _OUT_S), jnp.float32),
            jax.ShapeDtypeStruct((_N, 3 * _OUT_V), jnp.float32),
        ],
    )(s_part, s_part, v_part, v_part, Ws)


def kernel(node_attrs, positions, edge_index, shifts, W1, b1, W2s, W2v, Wv, Ws):
    sender = edge_index[0]
    receiver = edge_index[1]
    zero_s = jnp.zeros((64, _F), jnp.float32)
    epack = sender | (receiver << 16)
    p16 = jnp.pad(positions, ((0, 0), (0, 13)))

    xv = _run_prep(node_attrs, Wv)
    xvp = jnp.pad(xv, ((0, _NP - _N), (0, 0))).reshape(_NP // 4, 4 * _OUT_V)
    d = _run_geo(p16, sender, receiver)
    ws, wvn = _run_edge_dense(d, W1, b1, W2s, W2v)
    s_part = _run_scat_s(node_attrs, ws, sender, receiver, zero_s)
    v_part = _run_scat_v(xvp, wvn, epack)
    out_s, aggv_km = _run_final(s_part, v_part, Ws)

    agg_v = aggv_km.reshape(_N, 3, _OUT_V).transpose(0, 2, 1).reshape(_N, 3 * _OUT_V)
    return jnp.concatenate([out_s, agg_v], axis=1)
